# baseline (speedup 1.0000x reference)
import functools

import jax
import jax.numpy as jnp
from jax.experimental import pallas as pl
from jax.experimental.pallas import tpu as pltpu

_L_BLK = 512
_ROUNDS = 4
_NB2 = 32


def _lsh_body(inp_ref, rm_ref, out_ref, *, length):
    l = pl.program_id(1)
    x = inp_ref[0]
    rm = rm_ref[0]
    xn = x / jnp.maximum(
        jnp.sqrt(jnp.sum(x * x, axis=1, keepdims=True)), 1e-12)
    rmn = rm / jnp.sqrt(jnp.sum(rm * rm, axis=0, keepdims=True))
    m = jnp.dot(xn.astype(jnp.bfloat16), rmn.astype(jnp.bfloat16),
                preferred_element_type=jnp.float32)

    lane = jax.lax.broadcasted_iota(jnp.int32, (_L_BLK, _ROUNDS * _NB2), 1)
    bucket = lane % _NB2
    rnd = lane // _NB2
    tok = jax.lax.broadcasted_iota(jnp.int32, (_L_BLK, 1), 0) + l * _L_BLK

    pinf = jnp.float32(jnp.inf)
    cols = []
    for r in range(_ROUNDS):
        msk = rnd == r
        vmax = jnp.max(jnp.where(msk, m, -pinf), axis=1, keepdims=True)
        vmin = jnp.min(jnp.where(msk, m, pinf), axis=1, keepdims=True)
        imax = jnp.min(jnp.where(msk & (m == vmax), bucket, _NB2 * 2),
                       axis=1, keepdims=True)
        imin = jnp.min(jnp.where(msk & (m == vmin), bucket, _NB2 * 2),
                       axis=1, keepdims=True)
        h = jnp.where(vmax >= -vmin, imax, imin + _NB2)
        cols.append(h * length + tok)
    out_ref[0] = jnp.concatenate(cols, axis=1)


def kernel(inp, rand_matrix, n_buckets):
    del n_buckets
    batch, length, d_k = inp.shape
    rounds, nb2 = rand_matrix.shape[2], rand_matrix.shape[3]
    rm2 = rand_matrix.reshape(batch, d_k, rounds * nb2)
    grid = (batch, length // _L_BLK)
    return pl.pallas_call(
        functools.partial(_lsh_body, length=length),
        grid=grid,
        in_specs=[
            pl.BlockSpec((1, _L_BLK, d_k), lambda b, l: (b, l, 0)),
            pl.BlockSpec((1, d_k, rounds * nb2), lambda b, l: (b, 0, 0)),
        ],
        out_specs=pl.BlockSpec((1, _L_BLK, rounds), lambda b, l: (b, l, 0)),
        out_shape=jax.ShapeDtypeStruct((batch, length, rounds), jnp.int32),
        compiler_params=pltpu.CompilerParams(
            dimension_semantics=("arbitrary", "arbitrary"),
        ),
    )(inp, rm2)


# speedup vs baseline: 5.7211x; 5.7211x over previous
import functools

import jax
import jax.numpy as jnp
from jax.experimental import pallas as pl
from jax.experimental.pallas import tpu as pltpu

_L_BLK = 4096
_ROUNDS = 4
_NB2 = 32


def _lsh_body(inp_ref, rm_ref, out_ref, rmn_ref, *, length):
    l = pl.program_id(1)

    @pl.when(l == 0)
    def _():
        rmT = rm_ref[0]
        rmn = rmT / jnp.sqrt(jnp.sum(rmT * rmT, axis=1, keepdims=True))
        rmn_ref[...] = rmn.astype(jnp.bfloat16)

    x = inp_ref[0]
    ss = jnp.sum(x * x, axis=1)
    nrm = jnp.maximum(jnp.sqrt(ss), 1e-12)
    xn = x / nrm[:, None]
    xb = xn.astype(jnp.bfloat16)
    mT = jax.lax.dot_general(
        rmn_ref[...], xb,
        dimension_numbers=(((1,), (1,)), ((), ())),
        preferred_element_type=jnp.float32)

    av = jnp.abs(mT)
    rows = jax.lax.broadcasted_iota(jnp.int32, (_ROUNDS * _NB2, _L_BLK), 0)
    key_all = (rows % _NB2) + jnp.where(mT < 0, _NB2, 0)
    tok = jax.lax.broadcasted_iota(jnp.int32, (1, _L_BLK), 1) + l * _L_BLK
    cols = []
    for r in range(_ROUNDS):
        ar = av[r * _NB2:(r + 1) * _NB2]
        amax = jnp.max(ar, axis=0, keepdims=True)
        key = jnp.where(ar == amax, key_all[r * _NB2:(r + 1) * _NB2],
                        2 * _NB2)
        h = jnp.min(key, axis=0, keepdims=True)
        cols.append(h * length + tok)
    out_ref[0] = jnp.concatenate(cols, axis=0)


def kernel(inp, rand_matrix, n_buckets):
    del n_buckets
    batch, length, d_k = inp.shape
    rounds, nb2 = rand_matrix.shape[2], rand_matrix.shape[3]
    rmT = rand_matrix.transpose(0, 2, 3, 1).reshape(batch, rounds * nb2, d_k)
    grid = (batch, length // _L_BLK)
    out = pl.pallas_call(
        functools.partial(_lsh_body, length=length),
        grid=grid,
        in_specs=[
            pl.BlockSpec((1, _L_BLK, d_k), lambda b, l: (b, l, 0)),
            pl.BlockSpec((1, rounds * nb2, d_k), lambda b, l: (b, 0, 0)),
        ],
        out_specs=pl.BlockSpec((1, rounds, _L_BLK), lambda b, l: (b, 0, l)),
        out_shape=jax.ShapeDtypeStruct((batch, rounds, length), jnp.int32),
        scratch_shapes=[pltpu.VMEM((rounds * nb2, d_k), jnp.bfloat16)],
        compiler_params=pltpu.CompilerParams(
            dimension_semantics=("arbitrary", "arbitrary"),
        ),
    )(inp, rmT)
    return out.swapaxes(1, 2)
